# baseline (speedup 1.0000x reference)
import functools

import jax
import jax.numpy as jnp
from jax import lax
from jax.experimental import pallas as pl
from jax.experimental.pallas import tpu as pltpu
from jax.experimental.pallas import tpu_sc as plsc

_T_TILE = 256


def _vq_body(bs, n_tokens,
             param_ref, z_ref, cb_ref,
             idx_ref, loss_ref, perp_ref,
             csq_ref, counts_ref, kd_ref, ms_ref):
    i = pl.program_id(0)
    nt = pl.num_programs(0)
    cb = cb_ref[...]
    dim_z = cb.shape[1]
    K = cb.shape[0]

    @pl.when(i == 0)
    def _init():
        csq_ref[...] = jnp.sum(cb * cb, axis=1)[None, :]
        counts_ref[...] = jnp.zeros_like(counts_ref)
        kd_ref[0, 0] = 0.0
        ms_ref[0, 0] = 0.0

    w = 0.5 * (1.0 / jnp.clip(param_ref[0], 1e-10))
    z = z_ref[...]
    ntok = z.shape[0]
    zsq = jnp.sum(z * z, axis=1, keepdims=True)
    dots = lax.dot_general(z, cb, (((1,), (1,)), ((), ())),
                           preferred_element_type=jnp.float32)
    d = (zsq + csq_ref[...]) - 2.0 * dots

    dmin = jnp.min(d, axis=1)
    eq = d == dmin[:, None]
    onehotf = jnp.where(eq, 1.0, 0.0)

    rr = lax.broadcasted_iota(jnp.int32, (3, K), 1)
    cc = lax.broadcasted_iota(jnp.int32, (3, K), 0)
    digits = jnp.where(
        cc == 0, rr // 32, jnp.where(cc == 1, rr % 32, 1)
    ).astype(jnp.float32)
    pk = lax.dot_general(onehotf, digits, (((1,), (1,)), ((), ())),
                         preferred_element_type=jnp.float32)
    hi_s = pk[:, 0]
    lo_s = pk[:, 1]
    mlt = pk[:, 2]
    idxv = hi_s.astype(jnp.int32) * 32 + lo_s.astype(jnp.int32)
    idx_ref[0, 0, :] = idxv

    @pl.when(jnp.sum(mlt) != ntok)
    def _tie_fallback():
        iota = lax.broadcasted_iota(jnp.int32, d.shape, 1)
        idx_ref[0, 0, :] = jnp.min(jnp.where(eq, iota, K), axis=1)

    u = (dmin[:, None] - d) * w
    e = jnp.exp(u)
    onesk = jnp.ones((K, 1), jnp.float32)
    s = lax.dot_general(e, onesk, (((1,), (0,)), ((), ())),
                        preferred_element_type=jnp.float32)
    t = lax.dot_general(u * e, onesk, (((1,), (0,)), ((), ())),
                        preferred_element_type=jnp.float32)

    counts_ref[...] += lax.dot_general(
        jnp.ones((1, ntok), jnp.float32), onehotf,
        (((1,), (0,)), ((), ())), preferred_element_type=jnp.float32)
    kd_ref[0, 0] += jnp.sum(t / s - jnp.log(s))
    ms_ref[0, 0] += jnp.sum(-(w * dmin))

    @pl.when(i == nt - 1)
    def _fin():
        avg = counts_ref[...] * (1.0 / n_tokens)
        plogp = avg * jnp.log(avg + 1e-7)
        perp_ref[0, 0] = jnp.exp(-jnp.sum(plogp))
        loss_ref[0, 0] = (kd_ref[0, 0] - ms_ref[0, 0]) / bs


def _run_vq_main(param_q, z_flat, codebook, bs, interpret=False):
    n_tokens, dim_z = z_flat.shape
    K = codebook.shape[0]
    nt = n_tokens // _T_TILE
    body = functools.partial(_vq_body, bs, n_tokens)
    return pl.pallas_call(
        body,
        grid=(nt,),
        in_specs=[
            pl.BlockSpec(memory_space=pltpu.SMEM),
            pl.BlockSpec((_T_TILE, dim_z), lambda i: (i, 0)),
            pl.BlockSpec((K, dim_z), lambda i: (0, 0)),
        ],
        out_specs=[
            pl.BlockSpec((1, 1, _T_TILE), lambda i: (i, 0, 0)),
            pl.BlockSpec(memory_space=pltpu.SMEM),
            pl.BlockSpec(memory_space=pltpu.SMEM),
        ],
        out_shape=[
            jax.ShapeDtypeStruct((nt, 1, _T_TILE), jnp.int32),
            jax.ShapeDtypeStruct((1, 1), jnp.float32),
            jax.ShapeDtypeStruct((1, 1), jnp.float32),
        ],
        scratch_shapes=[
            pltpu.VMEM((1, K), jnp.float32),
            pltpu.VMEM((1, K), jnp.float32),
            pltpu.SMEM((1, 1), jnp.float32),
            pltpu.SMEM((1, 1), jnp.float32),
        ],
        interpret=interpret,
    )(param_q, z_flat, codebook)


def _sc_gather(codebook, idx):
    V, D = codebook.shape
    B = idx.shape[0]
    info = plsc.get_sparse_core_info()
    NW = info.num_cores * info.num_subcores
    b_per_w = B // NW
    mesh = plsc.VectorSubcoreMesh(core_axis_name="c", subcore_axis_name="s")

    @functools.partial(
        pl.kernel, mesh=mesh,
        out_type=jax.ShapeDtypeStruct((B, D), jnp.float32),
        scratch_types=[
            pltpu.VMEM((b_per_w,), jnp.int32),
            pltpu.VMEM((b_per_w, D), jnp.float32),
            pltpu.SemaphoreType.DMA,
        ],
    )
    def gk(cb_hbm, idx_hbm, out_hbm, idx_v, rows_v, sem):
        wid = lax.axis_index("s") * info.num_cores + lax.axis_index("c")
        base = wid * b_per_w
        pltpu.sync_copy(idx_hbm.at[pl.ds(base, b_per_w)], idx_v)
        pltpu.async_copy(cb_hbm.at[idx_v], rows_v, sem).wait()
        pltpu.sync_copy(rows_v, out_hbm.at[pl.ds(base, b_per_w)])

    return gk(codebook, idx)


def kernel(z_from_encoder, param_q, codebook, flg_train, flg_quant_det):
    bs, dim_z, width, height = z_from_encoder.shape
    n_tokens = bs * width * height
    z_flat = jnp.transpose(z_from_encoder, (0, 2, 3, 1)).reshape(
        n_tokens, dim_z)
    idx3, loss2, perp2 = _run_vq_main(
        param_q.reshape(1), z_flat, codebook, bs)
    idx = idx3.reshape(n_tokens)
    zq = _sc_gather(codebook, idx)
    z_to_decoder = jnp.transpose(
        zq.reshape(bs, width, height, dim_z), (0, 3, 1, 2))
    return z_to_decoder, loss2[0, 0], perp2[0, 0]


# speedup vs baseline: 1.2934x; 1.2934x over previous
import functools

import jax
import jax.numpy as jnp
from jax import lax
from jax.experimental import pallas as pl
from jax.experimental.pallas import tpu as pltpu
from jax.experimental.pallas import tpu_sc as plsc

_T_TILE = 256


def _vq_body(bs, n_tokens,
             param_ref, z_ref, cb_ref,
             idx_ref, loss_ref, perp_ref,
             csq_ref, counts_ref, kd_ref, ms_ref):
    i = pl.program_id(0)
    nt = pl.num_programs(0)
    cb = cb_ref[...]
    dim_z = cb.shape[1]
    K = cb.shape[0]

    @pl.when(i == 0)
    def _init():
        csq_ref[...] = jnp.sum(cb * cb, axis=1)[None, :]
        counts_ref[...] = jnp.zeros_like(counts_ref)
        kd_ref[0, 0] = 0.0
        ms_ref[0, 0] = 0.0

    w = 0.5 * (1.0 / jnp.clip(param_ref[0], 1e-10))
    z = z_ref[...]
    ntok = z.shape[0]
    zsq = jnp.sum(z * z, axis=1, keepdims=True)
    dots = lax.dot_general(z, cb, (((1,), (1,)), ((), ())),
                           preferred_element_type=jnp.float32)
    d = (zsq + csq_ref[...]) - 2.0 * dots

    dmin = jnp.min(d, axis=1)
    iota = lax.broadcasted_iota(jnp.int32, d.shape, 1)
    idxv = jnp.argmin(d, axis=1).astype(jnp.int32)
    onehotf = jnp.where(iota == idxv[:, None], 1.0, 0.0)
    idx_ref[0, 0, :] = idxv

    u = (dmin[:, None] - d) * w
    e = jnp.exp(u)
    onesk = jnp.ones((K, 1), jnp.float32)
    s = lax.dot_general(e, onesk, (((1,), (0,)), ((), ())),
                        preferred_element_type=jnp.float32)
    t = lax.dot_general(u * e, onesk, (((1,), (0,)), ((), ())),
                        preferred_element_type=jnp.float32)

    counts_ref[...] += lax.dot_general(
        jnp.ones((1, ntok), jnp.float32), onehotf,
        (((1,), (0,)), ((), ())), preferred_element_type=jnp.float32)
    kd_ref[0, 0] += jnp.sum(t / s - jnp.log(s))
    ms_ref[0, 0] += jnp.sum(-(w * dmin))

    @pl.when(i == nt - 1)
    def _fin():
        avg = counts_ref[...] * (1.0 / n_tokens)
        plogp = avg * jnp.log(avg + 1e-7)
        perp_ref[0, 0] = jnp.exp(-jnp.sum(plogp))
        loss_ref[0, 0] = (kd_ref[0, 0] - ms_ref[0, 0]) / bs


def _run_vq_main(param_q, z_flat, codebook, bs, interpret=False):
    n_tokens, dim_z = z_flat.shape
    K = codebook.shape[0]
    nt = n_tokens // _T_TILE
    body = functools.partial(_vq_body, bs, n_tokens)
    return pl.pallas_call(
        body,
        grid=(nt,),
        in_specs=[
            pl.BlockSpec(memory_space=pltpu.SMEM),
            pl.BlockSpec((_T_TILE, dim_z), lambda i: (i, 0)),
            pl.BlockSpec((K, dim_z), lambda i: (0, 0)),
        ],
        out_specs=[
            pl.BlockSpec((1, 1, _T_TILE), lambda i: (i, 0, 0)),
            pl.BlockSpec(memory_space=pltpu.SMEM),
            pl.BlockSpec(memory_space=pltpu.SMEM),
        ],
        out_shape=[
            jax.ShapeDtypeStruct((nt, 1, _T_TILE), jnp.int32),
            jax.ShapeDtypeStruct((1, 1), jnp.float32),
            jax.ShapeDtypeStruct((1, 1), jnp.float32),
        ],
        scratch_shapes=[
            pltpu.VMEM((1, K), jnp.float32),
            pltpu.VMEM((1, K), jnp.float32),
            pltpu.SMEM((1, 1), jnp.float32),
            pltpu.SMEM((1, 1), jnp.float32),
        ],
        interpret=interpret,
    )(param_q, z_flat, codebook)


def _sc_gather(codebook, idx):
    V, D = codebook.shape
    B = idx.shape[0]
    info = plsc.get_sparse_core_info()
    NW = info.num_cores * info.num_subcores
    b_per_w = B // NW
    mesh = plsc.VectorSubcoreMesh(core_axis_name="c", subcore_axis_name="s")

    @functools.partial(
        pl.kernel, mesh=mesh,
        out_type=jax.ShapeDtypeStruct((B, D), jnp.float32),
        scratch_types=[
            pltpu.VMEM((b_per_w,), jnp.int32),
            pltpu.VMEM((b_per_w, D), jnp.float32),
            pltpu.SemaphoreType.DMA,
        ],
    )
    def gk(cb_hbm, idx_hbm, out_hbm, idx_v, rows_v, sem):
        wid = lax.axis_index("s") * info.num_cores + lax.axis_index("c")
        base = wid * b_per_w
        pltpu.sync_copy(idx_hbm.at[pl.ds(base, b_per_w)], idx_v)
        pltpu.async_copy(cb_hbm.at[idx_v], rows_v, sem).wait()
        pltpu.sync_copy(rows_v, out_hbm.at[pl.ds(base, b_per_w)])

    return gk(codebook, idx)


def kernel(z_from_encoder, param_q, codebook, flg_train, flg_quant_det):
    bs, dim_z, width, height = z_from_encoder.shape
    n_tokens = bs * width * height
    z_flat = jnp.transpose(z_from_encoder, (0, 2, 3, 1)).reshape(
        n_tokens, dim_z)
    idx3, loss2, perp2 = _run_vq_main(
        param_q.reshape(1), z_flat, codebook, bs)
    idx = idx3.reshape(n_tokens)
    zq = _sc_gather(codebook, idx)
    z_to_decoder = jnp.transpose(
        zq.reshape(bs, width, height, dim_z), (0, 3, 1, 2))
    return z_to_decoder, loss2[0, 0], perp2[0, 0]


# speedup vs baseline: 1.3040x; 1.0082x over previous
import functools

import jax
import jax.numpy as jnp
from jax import lax
from jax.experimental import pallas as pl
from jax.experimental.pallas import tpu as pltpu
from jax.experimental.pallas import tpu_sc as plsc

_T_TILE = 256


def _vq_body(bs, n_tokens,
             param_ref, z_ref, cb_ref,
             idx_ref, loss_ref, perp_ref,
             csq_ref, counts_ref, kd_ref, ms_ref):
    i = pl.program_id(0)
    nt = pl.num_programs(0)
    cb = cb_ref[...]
    dim_z = cb.shape[1]
    K = cb.shape[0]

    @pl.when(i == 0)
    def _init():
        csq_ref[...] = jnp.sum(cb * cb, axis=1)[None, :]
        counts_ref[...] = jnp.zeros_like(counts_ref)
        kd_ref[0, 0] = 0.0
        ms_ref[0, 0] = 0.0

    w = 0.5 * (1.0 / jnp.clip(param_ref[0], 1e-10))
    z = z_ref[...]
    ntok = z.shape[0]
    zsq = jnp.sum(z * z, axis=1, keepdims=True)
    dots = lax.dot_general(z, cb, (((1,), (1,)), ((), ())),
                           preferred_element_type=jnp.float32)
    d = (zsq + csq_ref[...]) - 2.0 * dots

    dmin = jnp.min(d, axis=1)
    iota = lax.broadcasted_iota(jnp.int32, d.shape, 1)
    eq = d == dmin[:, None]
    onehotf = jnp.where(eq, 1.0, 0.0)
    idxv = jnp.min(jnp.where(eq, iota, K), axis=1)
    idx_ref[0, 0, :] = idxv

    u = (dmin[:, None] - d) * w
    e = jnp.exp(u)
    onesk = jnp.ones((K, 1), jnp.float32)
    s = lax.dot_general(e, onesk, (((1,), (0,)), ((), ())),
                        preferred_element_type=jnp.float32)
    t = lax.dot_general(u * e, onesk, (((1,), (0,)), ((), ())),
                        preferred_element_type=jnp.float32)

    counts_ref[...] += lax.dot_general(
        jnp.ones((1, ntok), jnp.float32), onehotf,
        (((1,), (0,)), ((), ())), preferred_element_type=jnp.float32)
    kd_ref[0, 0] += jnp.sum(t / s - jnp.log(s))
    ms_ref[0, 0] += jnp.sum(-(w * dmin))

    @pl.when(i == nt - 1)
    def _fin():
        avg = counts_ref[...] * (1.0 / n_tokens)
        plogp = avg * jnp.log(avg + 1e-7)
        perp_ref[0, 0] = jnp.exp(-jnp.sum(plogp))
        loss_ref[0, 0] = (kd_ref[0, 0] - ms_ref[0, 0]) / bs


def _run_vq_main(param_q, z_flat, codebook, bs, interpret=False):
    n_tokens, dim_z = z_flat.shape
    K = codebook.shape[0]
    nt = n_tokens // _T_TILE
    body = functools.partial(_vq_body, bs, n_tokens)
    return pl.pallas_call(
        body,
        grid=(nt,),
        in_specs=[
            pl.BlockSpec(memory_space=pltpu.SMEM),
            pl.BlockSpec((_T_TILE, dim_z), lambda i: (i, 0)),
            pl.BlockSpec((K, dim_z), lambda i: (0, 0)),
        ],
        out_specs=[
            pl.BlockSpec((1, 1, _T_TILE), lambda i: (i, 0, 0)),
            pl.BlockSpec(memory_space=pltpu.SMEM),
            pl.BlockSpec(memory_space=pltpu.SMEM),
        ],
        out_shape=[
            jax.ShapeDtypeStruct((nt, 1, _T_TILE), jnp.int32),
            jax.ShapeDtypeStruct((1, 1), jnp.float32),
            jax.ShapeDtypeStruct((1, 1), jnp.float32),
        ],
        scratch_shapes=[
            pltpu.VMEM((1, K), jnp.float32),
            pltpu.VMEM((1, K), jnp.float32),
            pltpu.SMEM((1, 1), jnp.float32),
            pltpu.SMEM((1, 1), jnp.float32),
        ],
        interpret=interpret,
    )(param_q, z_flat, codebook)


def _sc_gather(codebook, idx):
    V, D = codebook.shape
    B = idx.shape[0]
    info = plsc.get_sparse_core_info()
    NW = info.num_cores * info.num_subcores
    b_per_w = B // NW
    mesh = plsc.VectorSubcoreMesh(core_axis_name="c", subcore_axis_name="s")

    @functools.partial(
        pl.kernel, mesh=mesh,
        out_type=jax.ShapeDtypeStruct((B, D), jnp.float32),
        scratch_types=[
            pltpu.VMEM((b_per_w,), jnp.int32),
            pltpu.VMEM((b_per_w, D), jnp.float32),
            pltpu.SemaphoreType.DMA,
        ],
    )
    def gk(cb_hbm, idx_hbm, out_hbm, idx_v, rows_v, sem):
        wid = lax.axis_index("s") * info.num_cores + lax.axis_index("c")
        base = wid * b_per_w
        pltpu.sync_copy(idx_hbm.at[pl.ds(base, b_per_w)], idx_v)
        pltpu.async_copy(cb_hbm.at[idx_v], rows_v, sem).wait()
        pltpu.sync_copy(rows_v, out_hbm.at[pl.ds(base, b_per_w)])

    return gk(codebook, idx)


def kernel(z_from_encoder, param_q, codebook, flg_train, flg_quant_det):
    bs, dim_z, width, height = z_from_encoder.shape
    n_tokens = bs * width * height
    z_flat = jnp.transpose(z_from_encoder, (0, 2, 3, 1)).reshape(
        n_tokens, dim_z)
    idx3, loss2, perp2 = _run_vq_main(
        param_q.reshape(1), z_flat, codebook, bs)
    idx = idx3.reshape(n_tokens)
    zq = _sc_gather(codebook, idx)
    z_to_decoder = jnp.transpose(
        zq.reshape(bs, width, height, dim_z), (0, 3, 1, 2))
    return z_to_decoder, loss2[0, 0], perp2[0, 0]


# speedup vs baseline: 1.4317x; 1.0979x over previous
import functools

import jax
import jax.numpy as jnp
from jax import lax
from jax.experimental import pallas as pl
from jax.experimental.pallas import tpu as pltpu
from jax.experimental.pallas import tpu_sc as plsc

_T_TILE = 512


def _vq_body(bs, n_tokens,
             param_ref, z_ref, cb_ref,
             idx_ref, loss_ref, perp_ref,
             csq_ref, counts_ref, kd_ref, ms_ref):
    i = pl.program_id(0)
    nt = pl.num_programs(0)
    cb = cb_ref[...]
    dim_z = cb.shape[1]
    K = cb.shape[0]

    @pl.when(i == 0)
    def _init():
        csq_ref[...] = jnp.sum(cb * cb, axis=1)[None, :]
        counts_ref[...] = jnp.zeros_like(counts_ref)
        kd_ref[0, 0] = 0.0
        ms_ref[0, 0] = 0.0

    w = 0.5 * (1.0 / jnp.clip(param_ref[0], 1e-10))
    z = z_ref[...]
    ntok = z.shape[0]
    zsq = jnp.sum(z * z, axis=1, keepdims=True)
    dots = lax.dot_general(z, cb, (((1,), (1,)), ((), ())),
                           preferred_element_type=jnp.float32)
    d = (zsq + csq_ref[...]) - 2.0 * dots

    dmin = jnp.min(d, axis=1)
    iota = lax.broadcasted_iota(jnp.int32, d.shape, 1)
    eq = d == dmin[:, None]
    onehotf = jnp.where(eq, 1.0, 0.0)
    idxv = jnp.min(jnp.where(eq, iota, K), axis=1)
    idx_ref[0, 0, :] = idxv

    u = (dmin[:, None] - d) * w
    e = jnp.exp(u)
    onesk = jnp.ones((K, 1), jnp.float32)
    s = lax.dot_general(e, onesk, (((1,), (0,)), ((), ())),
                        preferred_element_type=jnp.float32)
    t = lax.dot_general(u * e, onesk, (((1,), (0,)), ((), ())),
                        preferred_element_type=jnp.float32)

    counts_ref[...] += lax.dot_general(
        jnp.ones((1, ntok), jnp.float32), onehotf,
        (((1,), (0,)), ((), ())), preferred_element_type=jnp.float32)
    kd_ref[0, 0] += jnp.sum(t / s - jnp.log(s))
    ms_ref[0, 0] += jnp.sum(-(w * dmin))

    @pl.when(i == nt - 1)
    def _fin():
        avg = counts_ref[...] * (1.0 / n_tokens)
        plogp = avg * jnp.log(avg + 1e-7)
        perp_ref[0, 0] = jnp.exp(-jnp.sum(plogp))
        loss_ref[0, 0] = (kd_ref[0, 0] - ms_ref[0, 0]) / bs


def _run_vq_main(param_q, z_flat, codebook, bs, interpret=False):
    n_tokens, dim_z = z_flat.shape
    K = codebook.shape[0]
    nt = n_tokens // _T_TILE
    body = functools.partial(_vq_body, bs, n_tokens)
    return pl.pallas_call(
        body,
        grid=(nt,),
        in_specs=[
            pl.BlockSpec(memory_space=pltpu.SMEM),
            pl.BlockSpec((_T_TILE, dim_z), lambda i: (i, 0)),
            pl.BlockSpec((K, dim_z), lambda i: (0, 0)),
        ],
        out_specs=[
            pl.BlockSpec((1, 1, _T_TILE), lambda i: (i, 0, 0)),
            pl.BlockSpec(memory_space=pltpu.SMEM),
            pl.BlockSpec(memory_space=pltpu.SMEM),
        ],
        out_shape=[
            jax.ShapeDtypeStruct((nt, 1, _T_TILE), jnp.int32),
            jax.ShapeDtypeStruct((1, 1), jnp.float32),
            jax.ShapeDtypeStruct((1, 1), jnp.float32),
        ],
        scratch_shapes=[
            pltpu.VMEM((1, K), jnp.float32),
            pltpu.VMEM((1, K), jnp.float32),
            pltpu.SMEM((1, 1), jnp.float32),
            pltpu.SMEM((1, 1), jnp.float32),
        ],
        compiler_params=pltpu.CompilerParams(
            vmem_limit_bytes=128 * 1024 * 1024),
        interpret=interpret,
    )(param_q, z_flat, codebook)


def _sc_gather(codebook, idx):
    V, D = codebook.shape
    B = idx.shape[0]
    info = plsc.get_sparse_core_info()
    NW = info.num_cores * info.num_subcores
    b_per_w = B // NW
    mesh = plsc.VectorSubcoreMesh(core_axis_name="c", subcore_axis_name="s")

    @functools.partial(
        pl.kernel, mesh=mesh,
        out_type=jax.ShapeDtypeStruct((B, D), jnp.float32),
        scratch_types=[
            pltpu.VMEM((b_per_w,), jnp.int32),
            pltpu.VMEM((b_per_w, D), jnp.float32),
            pltpu.SemaphoreType.DMA,
        ],
    )
    def gk(cb_hbm, idx_hbm, out_hbm, idx_v, rows_v, sem):
        wid = lax.axis_index("s") * info.num_cores + lax.axis_index("c")
        base = wid * b_per_w
        pltpu.sync_copy(idx_hbm.at[pl.ds(base, b_per_w)], idx_v)
        pltpu.async_copy(cb_hbm.at[idx_v], rows_v, sem).wait()
        pltpu.sync_copy(rows_v, out_hbm.at[pl.ds(base, b_per_w)])

    return gk(codebook, idx)


def kernel(z_from_encoder, param_q, codebook, flg_train, flg_quant_det):
    bs, dim_z, width, height = z_from_encoder.shape
    n_tokens = bs * width * height
    z_flat = jnp.transpose(z_from_encoder, (0, 2, 3, 1)).reshape(
        n_tokens, dim_z)
    idx3, loss2, perp2 = _run_vq_main(
        param_q.reshape(1), z_flat, codebook, bs)
    idx = idx3.reshape(n_tokens)
    zq = _sc_gather(codebook, idx)
    z_to_decoder = jnp.transpose(
        zq.reshape(bs, width, height, dim_z), (0, 3, 1, 2))
    return z_to_decoder, loss2[0, 0], perp2[0, 0]


# speedup vs baseline: 1.4700x; 1.0268x over previous
import functools

import jax
import jax.numpy as jnp
from jax import lax
from jax.experimental import pallas as pl
from jax.experimental.pallas import tpu as pltpu
from jax.experimental.pallas import tpu_sc as plsc

_T_TILE = 1024


def _vq_body(bs, n_tokens,
             param_ref, z_ref, cb_ref,
             idx_ref, loss_ref, perp_ref,
             csq_ref, counts_ref, kd_ref, ms_ref):
    i = pl.program_id(0)
    nt = pl.num_programs(0)
    cb = cb_ref[...]
    dim_z = cb.shape[1]
    K = cb.shape[0]

    @pl.when(i == 0)
    def _init():
        csq_ref[...] = jnp.sum(cb * cb, axis=1)[None, :]
        counts_ref[...] = jnp.zeros_like(counts_ref)
        kd_ref[0, 0] = 0.0
        ms_ref[0, 0] = 0.0

    w = 0.5 * (1.0 / jnp.clip(param_ref[0], 1e-10))
    z = z_ref[...]
    ntok = z.shape[0]
    zsq = jnp.sum(z * z, axis=1, keepdims=True)
    dots = lax.dot_general(z, cb, (((1,), (1,)), ((), ())),
                           preferred_element_type=jnp.float32)
    d = (zsq + csq_ref[...]) - 2.0 * dots

    dmin = jnp.min(d, axis=1)
    iota = lax.broadcasted_iota(jnp.int32, d.shape, 1)
    eq = d == dmin[:, None]
    onehotf = jnp.where(eq, 1.0, 0.0)
    idxv = jnp.min(jnp.where(eq, iota, K), axis=1)
    idx_ref[0, 0, :] = idxv

    u = (dmin[:, None] - d) * w
    e = jnp.exp(u)
    onesk = jnp.ones((K, 1), jnp.float32)
    s = lax.dot_general(e, onesk, (((1,), (0,)), ((), ())),
                        preferred_element_type=jnp.float32)
    t = lax.dot_general(u * e, onesk, (((1,), (0,)), ((), ())),
                        preferred_element_type=jnp.float32)

    counts_ref[...] += lax.dot_general(
        jnp.ones((1, ntok), jnp.float32), onehotf,
        (((1,), (0,)), ((), ())), preferred_element_type=jnp.float32)
    kd_ref[0, 0] += jnp.sum(t / s - jnp.log(s))
    ms_ref[0, 0] += jnp.sum(-(w * dmin))

    @pl.when(i == nt - 1)
    def _fin():
        avg = counts_ref[...] * (1.0 / n_tokens)
        plogp = avg * jnp.log(avg + 1e-7)
        perp_ref[0, 0] = jnp.exp(-jnp.sum(plogp))
        loss_ref[0, 0] = (kd_ref[0, 0] - ms_ref[0, 0]) / bs


def _run_vq_main(param_q, z_flat, codebook, bs, interpret=False):
    n_tokens, dim_z = z_flat.shape
    K = codebook.shape[0]
    nt = n_tokens // _T_TILE
    body = functools.partial(_vq_body, bs, n_tokens)
    return pl.pallas_call(
        body,
        grid=(nt,),
        in_specs=[
            pl.BlockSpec(memory_space=pltpu.SMEM),
            pl.BlockSpec((_T_TILE, dim_z), lambda i: (i, 0)),
            pl.BlockSpec((K, dim_z), lambda i: (0, 0)),
        ],
        out_specs=[
            pl.BlockSpec((1, 1, _T_TILE), lambda i: (i, 0, 0)),
            pl.BlockSpec(memory_space=pltpu.SMEM),
            pl.BlockSpec(memory_space=pltpu.SMEM),
        ],
        out_shape=[
            jax.ShapeDtypeStruct((nt, 1, _T_TILE), jnp.int32),
            jax.ShapeDtypeStruct((1, 1), jnp.float32),
            jax.ShapeDtypeStruct((1, 1), jnp.float32),
        ],
        scratch_shapes=[
            pltpu.VMEM((1, K), jnp.float32),
            pltpu.VMEM((1, K), jnp.float32),
            pltpu.SMEM((1, 1), jnp.float32),
            pltpu.SMEM((1, 1), jnp.float32),
        ],
        compiler_params=pltpu.CompilerParams(
            vmem_limit_bytes=128 * 1024 * 1024),
        interpret=interpret,
    )(param_q, z_flat, codebook)


def _sc_gather(codebook, idx):
    V, D = codebook.shape
    B = idx.shape[0]
    info = plsc.get_sparse_core_info()
    NW = info.num_cores * info.num_subcores
    b_per_w = B // NW
    mesh = plsc.VectorSubcoreMesh(core_axis_name="c", subcore_axis_name="s")

    @functools.partial(
        pl.kernel, mesh=mesh,
        out_type=jax.ShapeDtypeStruct((B, D), jnp.float32),
        scratch_types=[
            pltpu.VMEM((b_per_w,), jnp.int32),
            pltpu.VMEM((b_per_w, D), jnp.float32),
            pltpu.SemaphoreType.DMA,
        ],
    )
    def gk(cb_hbm, idx_hbm, out_hbm, idx_v, rows_v, sem):
        wid = lax.axis_index("s") * info.num_cores + lax.axis_index("c")
        base = wid * b_per_w
        pltpu.sync_copy(idx_hbm.at[pl.ds(base, b_per_w)], idx_v)
        pltpu.async_copy(cb_hbm.at[idx_v], rows_v, sem).wait()
        pltpu.sync_copy(rows_v, out_hbm.at[pl.ds(base, b_per_w)])

    return gk(codebook, idx)


def kernel(z_from_encoder, param_q, codebook, flg_train, flg_quant_det):
    bs, dim_z, width, height = z_from_encoder.shape
    n_tokens = bs * width * height
    z_flat = jnp.transpose(z_from_encoder, (0, 2, 3, 1)).reshape(
        n_tokens, dim_z)
    idx3, loss2, perp2 = _run_vq_main(
        param_q.reshape(1), z_flat, codebook, bs)
    idx = idx3.reshape(n_tokens)
    zq = _sc_gather(codebook, idx)
    z_to_decoder = jnp.transpose(
        zq.reshape(bs, width, height, dim_z), (0, 3, 1, 2))
    return z_to_decoder, loss2[0, 0], perp2[0, 0]
